# baseline (speedup 1.0000x reference)
import functools

import jax
import jax.numpy as jnp
from jax import lax
from jax.experimental import pallas as pl
from jax.experimental.pallas import tpu as pltpu
from jax.experimental.pallas import tpu_sc as plsc

_DFF = 4096
_DMODEL = 1024
_K = 1024
_MB = 512
_NBLK = _DFF // _MB


def _bottom_k_mask(m):
    u = jax.lax.bitcast_convert_type(m, jnp.int32)
    k = jnp.int32(_K)

    def bs_body(_, carry):
        lo, hi = carry
        mid = lo + (hi - lo) // 2
        c = jnp.sum((u <= mid).astype(jnp.int32))
        take = c >= k
        return jnp.where(take, lo, mid + 1), jnp.where(take, mid, hi)

    _, p = jax.lax.fori_loop(
        0, 31, bs_body, (jnp.int32(0), jnp.int32(0x7F800000)))

    lt = u < p
    eq = u == p
    c_lt = jnp.sum(lt.astype(jnp.int32))
    need = k - c_lt

    e = eq.astype(jnp.int32)
    x = e
    s = 1
    while s < _MB:
        sh = jnp.concatenate([jnp.zeros((_NBLK, s), jnp.int32), x[:, :-s]],
                             axis=1)
        x = x + sh
        s *= 2
    row_tot = x[:, _MB - 1:_MB]
    y = row_tot
    s = 1
    while s < _NBLK:
        shy = jnp.concatenate([jnp.zeros((s, 1), jnp.int32), y[:-s, :]],
                              axis=0)
        y = y + shy
        s *= 2
    row_off = jnp.concatenate([jnp.zeros((1, 1), jnp.int32), y[:-1, :]],
                              axis=0)
    excl = (x - e) + row_off
    prune_eq = eq & (excl < need)
    keep = jnp.logical_not(jnp.logical_or(lt, prune_eq))
    return keep.astype(jnp.float32)


def _fused_body(w1_ref, w2_ref, f1_ref, f2_ref,
                maskout_ref, w1out_ref, w2out_ref,
                mags_s, mask_s):
    i = pl.program_id(0)

    @pl.when(i < _NBLK)
    def _mags_phase():
        w1 = w1_ref[pl.ds(i * _MB, _MB), :]
        w2 = w2_ref[:, pl.ds(i * _MB, _MB)]
        s1 = jnp.sum(w1 * w1, axis=1)
        s2 = jnp.sum(w2 * w2, axis=0)
        mags_s[pl.ds(i, 1), :] = (jnp.sqrt(s1) * jnp.sqrt(s2)).reshape(1, _MB)

    @pl.when(i == _NBLK)
    def _mask_phase():
        mask = _bottom_k_mask(mags_s[...])
        mask_s[...] = mask
        maskout_ref[...] = mask

    @pl.when(i >= _NBLK)
    def _blend_phase():
        j = i - _NBLK
        mrow = mask_s[pl.ds(j, 1), :]
        keep_r = mrow > 0.5
        w2blk = w2_ref[:, pl.ds(j * _MB, _MB)]
        w2out_ref[...] = jnp.where(keep_r, w2blk, f2_ref[...])

        ii = jax.lax.broadcasted_iota(jnp.int32, (_MB, _MB), 0)
        jj = jax.lax.broadcasted_iota(jnp.int32, (_MB, _MB), 1)
        m_b = jnp.broadcast_to(mrow, (_MB, _MB))
        mcol = jnp.sum(jnp.where(ii == jj, m_b, 0.0), axis=1, keepdims=True)
        keep_c = mcol > 0.5
        w1blk = w1_ref[pl.ds(j * _MB, _MB), :]
        w1out_ref[...] = jnp.where(keep_c, w1blk, f1_ref[...])


def _relu_body(x_ref, y_ref):
    y_ref[...] = jnp.maximum(x_ref[...], 0.0)


_XN = 2 * 4096 * _DMODEL
_NWORK = 32
_WELEM = _XN // _NWORK
_CHE = 32768
_NCHUNK = _WELEM // _CHE


def _relu_sc_body(x_hbm, y_hbm, buf0, buf1, si0, si1, so0, so1):
    wid = lax.axis_index("s") * 2 + lax.axis_index("c")
    wbase = wid * _WELEM
    bufs = (buf0, buf1)
    isems = (si0, si1)
    osems = (so0, so1)
    in_cp = [None] * _NCHUNK
    out_cp = [None] * _NCHUNK
    in_cp[0] = pltpu.async_copy(x_hbm.at[pl.ds(wbase, _CHE)], buf0, si0)
    for c in range(_NCHUNK):
        b = c % 2
        in_cp[c].wait()
        if c >= 1:
            out_cp[c - 1].wait()
        if c + 1 < _NCHUNK:
            in_cp[c + 1] = pltpu.async_copy(
                x_hbm.at[pl.ds(wbase + (c + 1) * _CHE, _CHE)],
                bufs[1 - b], isems[1 - b])
        buf = bufs[b]

        def body(r, _, buf=buf):
            base = r * 128
            for t in range(8):
                sl = pl.ds(base + t * 16, 16)
                buf[sl] = jnp.maximum(buf[sl], 0.0)
            return 0

        lax.fori_loop(0, _CHE // 128, body, 0)
        out_cp[c] = pltpu.async_copy(
            buf, y_hbm.at[pl.ds(wbase + c * _CHE, _CHE)], osems[b])
    out_cp[_NCHUNK - 1].wait()


def _relu_sc(x2):
    mesh = plsc.VectorSubcoreMesh(core_axis_name="c", subcore_axis_name="s",
                                  num_cores=2, num_subcores=16)
    fn = pl.kernel(
        _relu_sc_body,
        out_type=jax.ShapeDtypeStruct((_XN,), jnp.float32),
        mesh=mesh,
        scratch_types=[
            pltpu.VMEM((_CHE,), jnp.float32),
            pltpu.VMEM((_CHE,), jnp.float32),
            pltpu.SemaphoreType.DMA,
            pltpu.SemaphoreType.DMA,
            pltpu.SemaphoreType.DMA,
            pltpu.SemaphoreType.DMA,
        ],
    )
    return fn(x2)


def kernel(x, W1, W2, frozen1, frozen2, target1, target2):
    del target1, target2

    mask2d, W1_new, W2_new = pl.pallas_call(
        _fused_body,
        grid=(2 * _NBLK,),
        in_specs=[
            pl.BlockSpec((_DFF, _DMODEL), lambda i: (0, 0)),
            pl.BlockSpec((_DMODEL, _DFF), lambda i: (0, 0)),
            pl.BlockSpec((_MB, _DMODEL),
                         lambda i: (jnp.maximum(i - _NBLK, 0), 0)),
            pl.BlockSpec((_DMODEL, _MB),
                         lambda i: (0, jnp.maximum(i - _NBLK, 0))),
        ],
        out_specs=[
            pl.BlockSpec((_NBLK, _MB), lambda i: (0, 0)),
            pl.BlockSpec((_MB, _DMODEL),
                         lambda i: (jnp.maximum(i - _NBLK, 0), 0)),
            pl.BlockSpec((_DMODEL, _MB),
                         lambda i: (0, jnp.maximum(i - _NBLK, 0))),
        ],
        out_shape=[
            jax.ShapeDtypeStruct((_NBLK, _MB), jnp.float32),
            jax.ShapeDtypeStruct((_DFF, _DMODEL), jnp.float32),
            jax.ShapeDtypeStruct((_DMODEL, _DFF), jnp.float32),
        ],
        scratch_shapes=[
            pltpu.VMEM((_NBLK, _MB), jnp.float32),
            pltpu.VMEM((_NBLK, _MB), jnp.float32),
        ],
    )(W1, W2, frozen1, frozen2)

    mask = mask2d.reshape(_DFF)

    y = _relu_sc(x.reshape(_XN))

    return y.reshape(x.shape), W1_new, W2_new, mask


# speedup vs baseline: 1.8811x; 1.8811x over previous
import functools

import jax
import jax.numpy as jnp
from jax import lax
from jax.experimental import pallas as pl
from jax.experimental.pallas import tpu as pltpu
from jax.experimental.pallas import tpu_sc as plsc

_DFF = 4096
_DMODEL = 1024
_K = 1024
_MB = 512
_NBLK = _DFF // _MB


def _bottom_k_mask(m):
    u = jax.lax.bitcast_convert_type(m, jnp.int32)
    k = jnp.int32(_K)

    def bs_body(_, carry):
        lo, hi = carry
        mid = lo + (hi - lo) // 2
        c = jnp.sum((u <= mid).astype(jnp.int32))
        take = c >= k
        return jnp.where(take, lo, mid + 1), jnp.where(take, mid, hi)

    _, p = jax.lax.fori_loop(
        0, 31, bs_body, (jnp.int32(0), jnp.int32(0x7F800000)))

    lt = u < p
    eq = u == p
    c_lt = jnp.sum(lt.astype(jnp.int32))
    need = k - c_lt

    e = eq.astype(jnp.int32)
    x = e
    s = 1
    while s < _MB:
        sh = jnp.concatenate([jnp.zeros((_NBLK, s), jnp.int32), x[:, :-s]],
                             axis=1)
        x = x + sh
        s *= 2
    row_tot = x[:, _MB - 1:_MB]
    y = row_tot
    s = 1
    while s < _NBLK:
        shy = jnp.concatenate([jnp.zeros((s, 1), jnp.int32), y[:-s, :]],
                              axis=0)
        y = y + shy
        s *= 2
    row_off = jnp.concatenate([jnp.zeros((1, 1), jnp.int32), y[:-1, :]],
                              axis=0)
    excl = (x - e) + row_off
    prune_eq = eq & (excl < need)
    keep = jnp.logical_not(jnp.logical_or(lt, prune_eq))
    return keep.astype(jnp.float32)


def _fused_body(w1_ref, w2_ref, f1_ref, f2_ref,
                maskout_ref, w1out_ref, w2out_ref,
                mags_s, mask_s):
    i = pl.program_id(0)

    @pl.when(i < _NBLK)
    def _mags_phase():
        w1 = w1_ref[pl.ds(i * _MB, _MB), :]
        w2 = w2_ref[:, pl.ds(i * _MB, _MB)]
        s1 = jnp.sum(w1 * w1, axis=1)
        s2 = jnp.sum(w2 * w2, axis=0)
        mags_s[pl.ds(i, 1), :] = (jnp.sqrt(s1) * jnp.sqrt(s2)).reshape(1, _MB)

    @pl.when(i == _NBLK)
    def _mask_phase():
        mask = _bottom_k_mask(mags_s[...])
        mask_s[...] = mask
        maskout_ref[...] = mask

    @pl.when(i >= _NBLK)
    def _blend_phase():
        j = i - _NBLK
        mrow = mask_s[pl.ds(j, 1), :]
        keep_r = mrow > 0.5
        w2blk = w2_ref[:, pl.ds(j * _MB, _MB)]
        w2out_ref[...] = jnp.where(keep_r, w2blk, f2_ref[...])

        ii = jax.lax.broadcasted_iota(jnp.int32, (_MB, _MB), 0)
        jj = jax.lax.broadcasted_iota(jnp.int32, (_MB, _MB), 1)
        m_b = jnp.broadcast_to(mrow, (_MB, _MB))
        mcol = jnp.sum(jnp.where(ii == jj, m_b, 0.0), axis=1, keepdims=True)
        keep_c = mcol > 0.5
        w1blk = w1_ref[pl.ds(j * _MB, _MB), :]
        w1out_ref[...] = jnp.where(keep_c, w1blk, f1_ref[...])


def _relu_body(x_ref, y_ref):
    y_ref[...] = jnp.maximum(x_ref[...], 0.0)


_XROWS = 2 * 4096
_NWORK = 32
_WR = _XROWS // _NWORK
_CHR = 32
_NCHUNK = _WR // _CHR


def _relu_sc_body(x_hbm, y_hbm, buf0, buf1, si0, si1, so0, so1):
    wid = lax.axis_index("s") * 2 + lax.axis_index("c")
    wbase = wid * _WR
    bufs = (buf0, buf1)
    isems = (si0, si1)
    osems = (so0, so1)
    in_cp = [None] * _NCHUNK
    out_cp = [None] * _NCHUNK
    in_cp[0] = pltpu.async_copy(x_hbm.at[pl.ds(wbase, _CHR)], buf0, si0)
    for c in range(_NCHUNK):
        b = c % 2
        in_cp[c].wait()
        if c >= 1:
            out_cp[c - 1].wait()
        if c + 1 < _NCHUNK:
            in_cp[c + 1] = pltpu.async_copy(
                x_hbm.at[pl.ds(wbase + (c + 1) * _CHR, _CHR)],
                bufs[1 - b], isems[1 - b])
        buf = bufs[b]

        def body(r, _, buf=buf):
            colbase = (r % 8) * 128
            row = r // 8
            for t in range(8):
                sl = pl.ds(colbase + t * 16, 16)
                buf[row, sl] = jnp.maximum(buf[row, sl], 0.0)
            return 0

        lax.fori_loop(0, _CHR * 8, body, 0)
        out_cp[c] = pltpu.async_copy(
            buf, y_hbm.at[pl.ds(wbase + c * _CHR, _CHR)], osems[b])
    out_cp[_NCHUNK - 1].wait()


def _relu_sc(x2):
    mesh = plsc.VectorSubcoreMesh(core_axis_name="c", subcore_axis_name="s",
                                  num_cores=2, num_subcores=16)
    fn = pl.kernel(
        _relu_sc_body,
        out_type=jax.ShapeDtypeStruct((_XROWS, _DMODEL), jnp.float32),
        mesh=mesh,
        scratch_types=[
            pltpu.VMEM((_CHR, _DMODEL), jnp.float32),
            pltpu.VMEM((_CHR, _DMODEL), jnp.float32),
            pltpu.SemaphoreType.DMA,
            pltpu.SemaphoreType.DMA,
            pltpu.SemaphoreType.DMA,
            pltpu.SemaphoreType.DMA,
        ],
    )
    return fn(x2)


def kernel(x, W1, W2, frozen1, frozen2, target1, target2):
    del target1, target2

    mask2d, W1_new, W2_new = pl.pallas_call(
        _fused_body,
        grid=(2 * _NBLK,),
        in_specs=[
            pl.BlockSpec((_DFF, _DMODEL), lambda i: (0, 0)),
            pl.BlockSpec((_DMODEL, _DFF), lambda i: (0, 0)),
            pl.BlockSpec((_MB, _DMODEL),
                         lambda i: (jnp.maximum(i - _NBLK, 0), 0)),
            pl.BlockSpec((_DMODEL, _MB),
                         lambda i: (0, jnp.maximum(i - _NBLK, 0))),
        ],
        out_specs=[
            pl.BlockSpec((_NBLK, _MB), lambda i: (0, 0)),
            pl.BlockSpec((_MB, _DMODEL),
                         lambda i: (jnp.maximum(i - _NBLK, 0), 0)),
            pl.BlockSpec((_DMODEL, _MB),
                         lambda i: (0, jnp.maximum(i - _NBLK, 0))),
        ],
        out_shape=[
            jax.ShapeDtypeStruct((_NBLK, _MB), jnp.float32),
            jax.ShapeDtypeStruct((_DFF, _DMODEL), jnp.float32),
            jax.ShapeDtypeStruct((_DMODEL, _DFF), jnp.float32),
        ],
        scratch_shapes=[
            pltpu.VMEM((_NBLK, _MB), jnp.float32),
            pltpu.VMEM((_NBLK, _MB), jnp.float32),
        ],
    )(W1, W2, frozen1, frozen2)

    mask = mask2d.reshape(_DFF)

    y = _relu_sc(x.reshape(_XROWS, _DMODEL))

    return y.reshape(x.shape), W1_new, W2_new, mask


# speedup vs baseline: 1.9605x; 1.0422x over previous
import jax
import jax.numpy as jnp
from jax.experimental import pallas as pl
from jax.experimental.pallas import tpu as pltpu

_DFF = 4096
_DMODEL = 1024
_K = 1024
_MB = 512
_NBLK = _DFF // _MB
_BB = 256
_NBB = _DFF // _BB
_XROWS = 2 * 4096
_NSTEP = 32
_XB = _XROWS // _NSTEP


def _bottom_k_mask(m):
    u = jax.lax.bitcast_convert_type(m, jnp.int32)
    k = jnp.int32(_K)

    def bs_body(_, carry):
        lo, hi = carry
        mid = lo + (hi - lo) // 2
        c = jnp.sum((u <= mid).astype(jnp.int32))
        take = c >= k
        return jnp.where(take, lo, mid + 1), jnp.where(take, mid, hi)

    _, p = jax.lax.fori_loop(
        0, 31, bs_body, (jnp.int32(0), jnp.int32(0x7F800000)))

    lt = u < p
    eq = u == p
    c_lt = jnp.sum(lt.astype(jnp.int32))
    need = k - c_lt

    e = eq.astype(jnp.int32)
    x = e
    s = 1
    while s < _MB:
        sh = jnp.concatenate([jnp.zeros((_NBLK, s), jnp.int32), x[:, :-s]],
                             axis=1)
        x = x + sh
        s *= 2
    row_tot = x[:, _MB - 1:_MB]
    y = row_tot
    s = 1
    while s < _NBLK:
        shy = jnp.concatenate([jnp.zeros((s, 1), jnp.int32), y[:-s, :]],
                              axis=0)
        y = y + shy
        s *= 2
    row_off = jnp.concatenate([jnp.zeros((1, 1), jnp.int32), y[:-1, :]],
                              axis=0)
    excl = (x - e) + row_off
    prune_eq = eq & (excl < need)
    keep = jnp.logical_not(jnp.logical_or(lt, prune_eq))
    return keep.astype(jnp.float32)


def _fused_body(w1_ref, w2_ref, f1_ref, f2_ref, x_ref,
                maskout_ref, w1out_ref, w2out_ref, y_ref,
                mags_s, mask_s):
    i = pl.program_id(0)

    y_ref[...] = jnp.maximum(x_ref[...], 0.0)

    @pl.when(i < _NBLK)
    def _mags_phase():
        w1 = w1_ref[pl.ds(i * _MB, _MB), :]
        w2 = w2_ref[:, pl.ds(i * _MB, _MB)]
        s1 = jnp.sum(w1 * w1, axis=1)
        s2 = jnp.sum(w2 * w2, axis=0)
        mags_s[pl.ds(i, 1), :] = (jnp.sqrt(s1) * jnp.sqrt(s2)).reshape(1, _MB)

    @pl.when(i == _NBLK)
    def _mask_phase():
        mask = _bottom_k_mask(mags_s[...])
        mask_s[...] = mask
        maskout_ref[...] = mask

    @pl.when(jnp.logical_and(i >= _NBLK, i < _NBLK + _NBB))
    def _blend_phase():
        j = i - _NBLK
        mrow = mask_s[pl.ds(j // 2, 1), pl.ds((j % 2) * _BB, _BB)]
        keep_r = mrow > 0.5
        w2blk = w2_ref[:, pl.ds(j * _BB, _BB)]
        w2out_ref[...] = jnp.where(keep_r, w2blk, f2_ref[...])

        ii = jax.lax.broadcasted_iota(jnp.int32, (_BB, _BB), 0)
        jj = jax.lax.broadcasted_iota(jnp.int32, (_BB, _BB), 1)
        m_b = jnp.broadcast_to(mrow, (_BB, _BB))
        mcol = jnp.sum(jnp.where(ii == jj, m_b, 0.0), axis=1, keepdims=True)
        keep_c = mcol > 0.5
        w1blk = w1_ref[pl.ds(j * _BB, _BB), :]
        w1out_ref[...] = jnp.where(keep_c, w1blk, f1_ref[...])


def kernel(x, W1, W2, frozen1, frozen2, target1, target2):
    del target1, target2

    x2 = x.reshape(_XROWS, _DMODEL)

    def _bmap(i):
        return jnp.clip(i - _NBLK, 0, _NBB - 1)

    mask2d, W1_new, W2_new, y = pl.pallas_call(
        _fused_body,
        grid=(_NSTEP,),
        in_specs=[
            pl.BlockSpec((_DFF, _DMODEL), lambda i: (0, 0)),
            pl.BlockSpec((_DMODEL, _DFF), lambda i: (0, 0)),
            pl.BlockSpec((_BB, _DMODEL), lambda i: (_bmap(i), 0)),
            pl.BlockSpec((_DMODEL, _BB), lambda i: (0, _bmap(i))),
            pl.BlockSpec((_XB, _DMODEL), lambda i: (i, 0)),
        ],
        out_specs=[
            pl.BlockSpec((_NBLK, _MB), lambda i: (0, 0)),
            pl.BlockSpec((_BB, _DMODEL), lambda i: (_bmap(i), 0)),
            pl.BlockSpec((_DMODEL, _BB), lambda i: (0, _bmap(i))),
            pl.BlockSpec((_XB, _DMODEL), lambda i: (i, 0)),
        ],
        out_shape=[
            jax.ShapeDtypeStruct((_NBLK, _MB), jnp.float32),
            jax.ShapeDtypeStruct((_DFF, _DMODEL), jnp.float32),
            jax.ShapeDtypeStruct((_DMODEL, _DFF), jnp.float32),
            jax.ShapeDtypeStruct((_XROWS, _DMODEL), jnp.float32),
        ],
        scratch_shapes=[
            pltpu.VMEM((_NBLK, _MB), jnp.float32),
            pltpu.VMEM((_NBLK, _MB), jnp.float32),
        ],
    )(W1, W2, frozen1, frozen2, x2)

    mask = mask2d.reshape(_DFF)
    return y.reshape(x.shape), W1_new, W2_new, mask
